# Initial kernel scaffold; baseline (speedup 1.0000x reference)
import jax
import jax.numpy as jnp
from jax.experimental import pallas as pl


def kernel(x2d, projected_pix, fov_mask):
    raise NotImplementedError("write your pallas kernel here")


# speedup vs baseline: 1.5138x; 1.5138x over previous
import functools

import jax
import jax.numpy as jnp
from jax import lax
from jax.experimental import pallas as pl
from jax.experimental.pallas import tpu as pltpu
from jax.experimental.pallas import tpu_sc as plsc

_NC = 2
_NS = 16
_NW = _NC * _NS
_L = 16


def _gather_kernel(N, HW, C, H, W):
    b_per_w = N // _NW
    n_chunks = b_per_w // 128
    n_vec = b_per_w // _L

    mesh = plsc.VectorSubcoreMesh(core_axis_name="c", subcore_axis_name="s")

    @functools.partial(
        pl.kernel,
        mesh=mesh,
        out_type=jax.ShapeDtypeStruct((N, C), jnp.float32),
        scratch_types=[
            pltpu.VMEM((b_per_w,), jnp.int32),
            pltpu.VMEM((b_per_w,), jnp.int32),
            pltpu.VMEM((b_per_w,), jnp.int32),
            pltpu.VMEM((n_chunks, 128), jnp.int32),
            pltpu.VMEM((128, C), jnp.float32),
            pltpu.SemaphoreType.DMA,
        ],
    )
    def k(table_hbm, px_hbm, py_hbm, fov_hbm, out_hbm,
          px_v, py_v, fov_v, idx_v, rows_v, sem):
        wid = lax.axis_index("s") * _NC + lax.axis_index("c")
        base = wid * b_per_w

        pltpu.sync_copy(px_hbm.at[pl.ds(base, b_per_w)], px_v)
        pltpu.sync_copy(py_hbm.at[pl.ds(base, b_per_w)], py_v)
        pltpu.sync_copy(fov_hbm.at[pl.ds(base, b_per_w)], fov_v)

        def idx_body(j, _):
            xv = px_v[pl.ds(j * _L, _L)]
            yv = py_v[pl.ds(j * _L, _L)]
            fv = fov_v[pl.ds(j * _L, _L)]
            xc = jnp.clip(xv, 0, W - 1)
            yc = jnp.clip(yv, 0, H - 1)
            idx = jnp.where(fv > 0, yc * W + xc, HW)
            idx_v[j // 8, pl.ds((j % 8) * _L, _L)] = idx
            return 0

        lax.fori_loop(0, n_vec, idx_body, 0, unroll=8)

        def gather_body(j, _):
            pltpu.async_copy(table_hbm.at[idx_v.at[j]], rows_v, sem).wait()
            pltpu.sync_copy(rows_v, out_hbm.at[pl.ds(base + j * 128, 128)])
            return 0

        lax.fori_loop(0, n_chunks, gather_body, 0)

    return k


def kernel(x2d, projected_pix, fov_mask):
    bs, c, h, w = x2d.shape
    n = projected_pix.shape[1]
    hw = h * w

    table = jnp.concatenate(
        [x2d.reshape(c, hw).T, jnp.zeros((1, c), jnp.float32)], axis=0)
    px = projected_pix[0, :, 0]
    py = projected_pix[0, :, 1]
    fov = fov_mask[0].astype(jnp.int32)

    y = _gather_kernel(n, hw, c, h, w)(table, px, py, fov)

    sx, sy, sz = 128, 128, 16
    return y.T.reshape(bs, c, sx, sy, sz)


# speedup vs baseline: 21.6317x; 14.2901x over previous
import functools

import jax
import jax.numpy as jnp
from jax import lax
from jax.experimental import pallas as pl
from jax.experimental.pallas import tpu as pltpu
from jax.experimental.pallas import tpu_sc as plsc

_NC = 2
_NS = 16
_NW = _NC * _NS
_L = 16
_NPAD = 128
_NB = 4


def _gather_kernel(N, HW, C, H, W):
    b_per_w = N // _NW
    n_chunks = b_per_w // 128
    n_vec = b_per_w // _L

    mesh = plsc.VectorSubcoreMesh(core_axis_name="c", subcore_axis_name="s")

    @functools.partial(
        pl.kernel,
        mesh=mesh,
        out_type=jax.ShapeDtypeStruct((N, C), jnp.float32),
        scratch_types=[
            pltpu.VMEM((b_per_w,), jnp.int32),
            pltpu.VMEM((b_per_w,), jnp.int32),
            pltpu.VMEM((b_per_w,), jnp.int32),
            pltpu.VMEM((n_chunks, 128), jnp.int32),
        ]
        + [pltpu.VMEM((128, C), jnp.float32) for _ in range(_NB)]
        + [pltpu.SemaphoreType.DMA for _ in range(2 * _NB)],
    )
    def k(table_hbm, px_hbm, py_hbm, fov_hbm, out_hbm,
          px_v, py_v, fov_v, idx_v, *bufs_and_sems):
        bufs = bufs_and_sems[:_NB]
        gsems = bufs_and_sems[_NB:2 * _NB]
        wsems = bufs_and_sems[2 * _NB:]

        wid = lax.axis_index("s") * _NC + lax.axis_index("c")
        base = wid * b_per_w

        pltpu.sync_copy(px_hbm.at[pl.ds(base, b_per_w)], px_v)
        pltpu.sync_copy(py_hbm.at[pl.ds(base, b_per_w)], py_v)
        pltpu.sync_copy(fov_hbm.at[pl.ds(base, b_per_w)], fov_v)

        lane = lax.broadcasted_iota(jnp.int32, (_L,), 0)

        def idx_body(j, _):
            xv = px_v[pl.ds(j * _L, _L)]
            yv = py_v[pl.ds(j * _L, _L)]
            fv = fov_v[pl.ds(j * _L, _L)]
            xc = jnp.clip(xv, 0, W - 1)
            yc = jnp.clip(yv, 0, H - 1)
            pad = HW + (j % 8) * _L + lane
            idx = jnp.where(fv > 0, yc * W + xc, pad)
            idx_v[j // 8, pl.ds((j % 8) * _L, _L)] = idx
            return 0

        lax.fori_loop(0, n_vec, idx_body, 0, unroll=8)

        def g_start(j, s):
            pltpu.async_copy(table_hbm.at[idx_v.at[j]], bufs[s], gsems[s])

        def g_wait(j, s):
            pltpu.make_async_copy(
                table_hbm.at[idx_v.at[j]], bufs[s], gsems[s]).wait()

        def w_start(j, s):
            pltpu.async_copy(
                bufs[s], out_hbm.at[pl.ds(base + j * 128, 128)], wsems[s])

        def w_wait(j, s):
            pltpu.make_async_copy(
                bufs[s], out_hbm.at[pl.ds(base + j * 128, 128)],
                wsems[s]).wait()

        for j in range(_NB - 1):
            g_start(j, j)
        for g in range(n_chunks):
            s = g % _NB
            g_wait(g, s)
            w_start(g, s)
            jn = g + _NB - 1
            if jn < n_chunks:
                s2 = jn % _NB
                if g >= 1:
                    w_wait(g - 1, s2)
                g_start(jn, s2)
        for j in range(n_chunks - _NB, n_chunks):
            w_wait(j, j % _NB)

    return k


def kernel(x2d, projected_pix, fov_mask):
    bs, c, h, w = x2d.shape
    n = projected_pix.shape[1]
    hw = h * w

    table = jnp.concatenate(
        [x2d.reshape(c, hw).T, jnp.zeros((_NPAD, c), jnp.float32)], axis=0)
    px = projected_pix[0, :, 0]
    py = projected_pix[0, :, 1]
    fov = fov_mask[0].astype(jnp.int32)

    y = _gather_kernel(n, hw, c, h, w)(table, px, py, fov)

    sx, sy, sz = 128, 128, 16
    return y.T.reshape(bs, c, sx, sy, sz)
